# baseline (speedup 1.0000x reference)
import jax
import jax.numpy as jnp
from jax.experimental import pallas as pl
from jax.experimental.pallas import tpu as pltpu

N = 10000
D = 128
H = 128
G = 64
SCORE = 10


NC = 4
CN = N // NC


def _pfn_kernel(x_hbm, batch_ref, pw1_ref, pb1_ref, pw2_ref, pb2_ref,
                fw1_ref, fb1_ref, fw2_ref, fb2_ref, out_ref,
                xv_ref, sems):
    copies = [pltpu.make_async_copy(
        x_hbm.at[pl.ds(k * CN, CN), :], xv_ref.at[pl.ds(k * CN, CN), :],
        sems.at[k]) for k in range(NC)]
    for c in copies:
        c.start()
    p = jnp.zeros((G, H), jnp.float32)
    cnt = jnp.zeros((G, 1), jnp.float32)
    for k in range(NC):
        copies[k].wait()
        h1 = jax.lax.dot_general(xv_ref[pl.ds(k * CN, CN), :], pw1_ref[...],
                                 (((1,), (0,)), ((), ())),
                                 preferred_element_type=jnp.float32)
        h1 = jnp.maximum(h1 + pb1_ref[...], 0.0)
        onehot_t = (batch_ref[:, pl.ds(k * CN, CN)] ==
                    jax.lax.broadcasted_iota(jnp.int32, (G, 1), 0)
                    ).astype(jnp.float32)
        p += jax.lax.dot_general(onehot_t, h1, (((1,), (0,)), ((), ())),
                                 preferred_element_type=jnp.float32)
        cnt += jnp.sum(onehot_t, axis=1, keepdims=True)
    hp = jax.lax.Precision.HIGHEST
    pooled = jax.lax.dot_general(p, pw2_ref[...], (((1,), (0,)), ((), ())),
                                 preferred_element_type=jnp.float32,
                                 precision=hp)
    pooled = pooled + cnt * pb2_ref[...]
    z = jax.lax.dot_general(pooled, fw1_ref[...], (((1,), (0,)), ((), ())),
                            preferred_element_type=jnp.float32, precision=hp)
    z = jnp.maximum(z + fb1_ref[...], 0.0)
    out_ref[...] = jax.lax.dot_general(z, fw2_ref[...], (((1,), (0,)), ((), ())),
                                       preferred_element_type=jnp.float32,
                                       precision=hp) + fb2_ref[...]


@jax.jit
def _run(x, batch2d, phi_W1, phi_b1, phi_W2, phi_b2, f_W1, f_b1, f_W2, f_b2):
    vmem = pl.BlockSpec(memory_space=pltpu.MemorySpace.VMEM)
    return pl.pallas_call(
        _pfn_kernel,
        in_specs=[pl.BlockSpec(memory_space=pltpu.MemorySpace.HBM)] + [vmem] * 9,
        out_shape=jax.ShapeDtypeStruct((G, SCORE), jnp.float32),
        scratch_shapes=[pltpu.VMEM((N, D), jnp.float32),
                        pltpu.SemaphoreType.DMA((NC,))],
    )(x, batch2d, phi_W1, phi_b1.reshape(1, H), phi_W2, phi_b2.reshape(1, D),
      f_W1, f_b1.reshape(1, H), f_W2, f_b2.reshape(1, SCORE))


def kernel(x, edge_index, batch, phi_W1, phi_b1, phi_W2, phi_b2,
           f_W1, f_b1, f_W2, f_b2):
    del edge_index
    return _run(x, batch.reshape(1, N), phi_W1, phi_b1, phi_W2, phi_b2,
                f_W1, f_b1, f_W2, f_b2)
